# Initial kernel scaffold; baseline (speedup 1.0000x reference)
import jax
import jax.numpy as jnp
from jax.experimental import pallas as pl


def kernel(emb, Wk, bk, Wq, bq, Wv, bv, Wa, ba, Wm, bm, Wout, bout, user_ids, game_ids, edge_played, edge_rev):
    raise NotImplementedError("write your pallas kernel here")


# speedup vs baseline: 6.0727x; 6.0727x over previous
import functools
import jax
import jax.numpy as jnp
from jax import lax
from jax.experimental import pallas as pl
from jax.experimental.pallas import tpu as pltpu
from jax.experimental.pallas import tpu_sc as plsc

_L = 2
_T = 2
_N = 10000
_E = 160000
_D = 256
_H = 8
_REL = [(0, 1), (1, 0)]

_WH = 128
_NW = 32
_BE = 128
_EPW = _E // _NW
_NCH = 40
_EPS = _E // 16
_NCHA = 80
_NP = _N + 8
_RPT = 640
_RPT_LAST = _N - 15 * _RPT
_BM = 2000


def _proj_body(x_ref, w_ref, b_ref, y_ref):
    y_ref[0, 0] = (jnp.dot(x_ref[0], w_ref[0, 0],
                           preferred_element_type=jnp.float32) + b_ref[0, 0])


_proj = pl.pallas_call(
    _proj_body,
    grid=(2, _T, _N // _BM),
    in_specs=[
        pl.BlockSpec((1, _BM, _D), lambda h, t, i: (t, i, 0)),
        pl.BlockSpec((1, 1, _D, _WH), lambda h, t, i: (h, t, 0, 0)),
        pl.BlockSpec((1, 1, 1, _WH), lambda h, t, i: (h, t, 0, 0)),
    ],
    out_specs=pl.BlockSpec((1, 1, _BM, _WH), lambda h, t, i: (h, t, i, 0)),
    out_shape=jax.ShapeDtypeStruct((2, _T, _N, _WH), jnp.float32),
)


def _out_body(a_ref, b_ref, ca_ref, cb_ref, x_ref, wa_ref, wb_ref, bias_ref, y_ref):
    cnt = jnp.maximum(ca_ref[0, 0][:, 0:1] + cb_ref[0, 0][:, 0:1], 1.0)
    y = jnp.dot(a_ref[0, 0], wa_ref[0], preferred_element_type=jnp.float32)
    y = y + jnp.dot(b_ref[0, 0], wb_ref[0], preferred_element_type=jnp.float32)
    y = y / cnt + bias_ref[0] + x_ref[0]
    y_ref[0] = jnp.maximum(y, 0.0)


_outproj = pl.pallas_call(
    _out_body,
    grid=(_T, _N // _BM),
    in_specs=[
        pl.BlockSpec((1, 1, _BM, _WH), lambda t, i: (t, 0, i, 0)),
        pl.BlockSpec((1, 1, _BM, _WH), lambda t, i: (t, 1, i, 0)),
        pl.BlockSpec((1, 1, _BM, _WH), lambda t, i: (t, 0, i, 0)),
        pl.BlockSpec((1, 1, _BM, _WH), lambda t, i: (t, 1, i, 0)),
        pl.BlockSpec((1, _BM, _D), lambda t, i: (t, i, 0)),
        pl.BlockSpec((1, _WH, _D), lambda t, i: (t, 0, 0)),
        pl.BlockSpec((1, _WH, _D), lambda t, i: (t, 0, 0)),
        pl.BlockSpec((1, 1, _D), lambda t, i: (t, 0, 0)),
    ],
    out_specs=pl.BlockSpec((1, _BM, _D), lambda t, i: (t, i, 0)),
    out_shape=jax.ShapeDtypeStruct((_T, _N, _D), jnp.float32),
)


def _rowrange_copy(s, src_at, dst_at, last=_RPT_LAST):
    base = pl.multiple_of(s * _RPT, 8)

    @pl.when(s < 15)
    def _():
        pltpu.sync_copy(src_at(pl.ds(base, _RPT)), dst_at(pl.ds(base, _RPT)))

    @pl.when(s == 15)
    def _():
        pltpu.sync_copy(src_at(pl.ds(15 * _RPT, last)),
                        dst_at(pl.ds(15 * _RPT, last)))


def _sc_agg_body(m2, sw0, dw0, sw1, dw1, zeros_hbm, out,
                 si_v, di_v, r0, r1, acc, semA, semB):
    c = lax.axis_index("c")
    s = lax.axis_index("s")

    def start(j, rbuf, sem):
        pltpu.async_copy(m2.at[si_v.at[j]], rbuf, sem)

    def wait(rbuf, sem):
        pltpu.make_async_copy(m2.at[si_v.at[0]], rbuf, sem).wait()

    def scat(j, rbuf):
        pltpu.sync_copy(rbuf, acc.at[di_v.at[j]], add=True)

    ncp = _NCHA // 2

    for sw2, dwr, dt in ((sw0, dw0, 1), (sw1, dw1, 0)):
        _rowrange_copy(s, lambda d: zeros_hbm.at[d], lambda d: acc.at[d],
                       last=_NP - 15 * _RPT)

        for p in range(2):
            pltpu.sync_copy(sw2.at[c, s, pl.ds(p * ncp, ncp)], si_v)
            pltpu.sync_copy(dwr.at[s, pl.ds(p * ncp, ncp)], di_v)
            if p == 0:
                plsc.subcore_barrier()

            start(0, r0, semA)

            def body(jj, carry):
                a = jj * 2
                start(a + 1, r1, semB)
                wait(r0, semA)
                scat(a, r0)

                @pl.when(a + 2 < ncp)
                def _():
                    start(a + 2, r0, semA)

                wait(r1, semB)
                scat(a + 1, r1)
                return carry

            lax.fori_loop(0, ncp // 2, body, 0)

        plsc.subcore_barrier()
        _rowrange_copy(s, lambda d: acc.at[d], lambda d: out.at[dt, c, d])
        plsc.subcore_barrier()


def _sc_cnt_body(dw0, dw1, ones_hbm, zeros_hbm, out,
                 di_v, ones_v, acc, sem):
    c = lax.axis_index("c")
    s = lax.axis_index("s")
    w = s * 2 + c

    pltpu.sync_copy(ones_hbm, ones_v)

    for dwr, dt in ((dw0, 1), (dw1, 0)):
        _rowrange_copy(s, lambda d: zeros_hbm.at[d], lambda d: acc.at[d],
                       last=_NP - 15 * _RPT)
        pltpu.sync_copy(dwr.at[w], di_v)
        plsc.subcore_barrier()

        def fire(j, carry):
            pltpu.sync_copy(ones_v, acc.at[di_v.at[j]], add=True)
            return carry

        lax.fori_loop(0, _NCH, fire, 0)
        plsc.subcore_barrier()
        _rowrange_copy(s, lambda d: acc.at[d], lambda d: out.at[dt, c, d])
        plsc.subcore_barrier()


@functools.lru_cache(maxsize=1)
def _get_sc_kernels():
    mesh = plsc.VectorSubcoreMesh(core_axis_name="c", subcore_axis_name="s")
    agg = functools.partial(
        pl.kernel,
        mesh=mesh,
        out_type=jax.ShapeDtypeStruct((_T, 2, _N, _WH), jnp.float32),
        scratch_types=[
            pltpu.VMEM((_NCHA // 2, _BE), jnp.int32),
            pltpu.VMEM((_NCHA // 2, _BE), jnp.int32),
            pltpu.VMEM((_BE, _WH), jnp.float32),
            pltpu.VMEM((_BE, _WH), jnp.float32),
            pltpu.VMEM_SHARED((_NP, _WH), jnp.float32),
            pltpu.SemaphoreType.DMA,
            pltpu.SemaphoreType.DMA,
        ],
    )(_sc_agg_body)
    cnt = functools.partial(
        pl.kernel,
        mesh=mesh,
        out_type=jax.ShapeDtypeStruct((_T, 2, _N, _WH), jnp.float32),
        scratch_types=[
            pltpu.VMEM((_NCH, _BE), jnp.int32),
            pltpu.VMEM((_BE, _WH), jnp.float32),
            pltpu.VMEM_SHARED((_NP, _WH), jnp.float32),
            pltpu.SemaphoreType.DMA,
        ],
    )(_sc_cnt_body)
    return agg, cnt


def kernel(emb, Wk, bk, Wq, bq, Wv, bv, Wa, ba, Wm, bm, Wout, bout,
           user_ids, game_ids, edge_played, edge_rev):
    f32 = jnp.float32
    x0 = jnp.take(emb[0], user_ids, axis=0)
    x1 = jnp.take(emb[1], game_ids, axis=0)
    X = jnp.stack([x0, x1])

    edges = [edge_played, edge_rev]
    npad_a = _NCHA * _BE - _EPS
    npad_c = _NCH * _BE - _EPW
    sw, dw, dwc = [], [], []
    for r, (st, dt) in enumerate(_REL):
        base = (edges[r][0] + st * _N).astype(jnp.int32).reshape(16, _EPS)
        base = jnp.pad(base, ((0, 0), (0, npad_a)))
        sw.append(jnp.stack([base, base + 2 * _N]).reshape(2, 16, _NCHA, _BE))
        di = edges[r][1].astype(jnp.int32)
        di_a = jnp.pad(di.reshape(16, _EPS), ((0, 0), (0, npad_a)),
                       constant_values=_N)
        dw.append(di_a.reshape(16, _NCHA, _BE))
        di_c = jnp.pad(di.reshape(_NW, _EPW), ((0, 0), (0, npad_c)),
                       constant_values=_N)
        dwc.append(di_c.reshape(_NW, _NCH, _BE))

    zeros = jnp.zeros((_NP, _WH), f32)
    ones_rows = jnp.ones((_BE, _WH), f32)
    inv_h = f32(1.0 / _H)

    sc_agg, sc_cnt = _get_sc_kernels()

    cnt_all = sc_cnt(dwc[0], dwc[1], ones_rows, zeros)

    for l in range(_L):
        Wf = jnp.einsum("tde,tef->tdf", Wv[l], Wm[l]) * inv_h
        bf = (jnp.einsum("te,tef->tf", bv[l], Wm[l]) + bm[l]) * inv_h
        Wst = jnp.stack([Wf[:, :, :_WH], Wf[:, :, _WH:]])
        bst = jnp.stack([bf[:, None, :_WH], bf[:, None, _WH:]])

        M = _proj(X, Wst, bst)
        m2 = M.reshape(2 * _T * _N, _WH)

        agg_all = sc_agg(m2, sw[0], dw[0], sw[1], dw[1], zeros)

        X = _outproj(agg_all, agg_all, cnt_all, cnt_all, X,
                     Wout[l, :, :_WH, :], Wout[l, :, _WH:, :],
                     bout[l][:, None, :])
    return X
